# baseline (speedup 1.0000x reference)
import functools

import jax
import jax.numpy as jnp
from jax import lax
from jax.experimental import pallas as pl
from jax.experimental.pallas import tpu as pltpu
from jax.experimental.pallas import tpu_sc as plsc

N = 10000
D = 128
E = 320000

NC = 2
NS = 16
CH = 128
EPT = E // NS
EPT_P = 20480
NCHUNK = EPT_P // CH
RPT = 632
NRELOAD = 5
IBUF = NCHUNK // NRELOAD
NJUNK = N
NA = N + 16

_mesh = plsc.VectorSubcoreMesh(core_axis_name="c", subcore_axis_name="s")


@functools.partial(
    pl.kernel,
    out_type=jax.ShapeDtypeStruct((NC, N, D), jnp.float32),
    mesh=_mesh,
    scratch_types=[
        pltpu.VMEM_SHARED((NA, D), jnp.float32),
        pltpu.VMEM((IBUF, CH), jnp.int32),
        pltpu.VMEM((CH, D), jnp.float32),
        pltpu.VMEM((8, D), jnp.float32),
    ],
)
def _sc_degree(dsts_hbm, deg_hbm, acc_sh, dst_all, ones_v, zbuf):
    cid = lax.axis_index("c")
    sid = lax.axis_index("s")
    row0 = jnp.minimum(sid * RPT, N - RPT)

    def initbuf(i, _):
        def initrow(j, __):
            zbuf[i, pl.ds(j * 16, 16)] = jnp.zeros((16,), jnp.float32)
            return __
        lax.fori_loop(0, D // 16, initrow, None)
        return _
    lax.fori_loop(0, 8, initbuf, None)

    def initones(i, _):
        def initrow(j, __):
            ones_v[i, pl.ds(j * 16, 16)] = jnp.ones((16,), jnp.float32)
            return __
        lax.fori_loop(0, D // 16, initrow, None)
        return _
    lax.fori_loop(0, CH, initones, None)

    def zrow(k, _):
        pltpu.sync_copy(zbuf, acc_sh.at[pl.ds(row0 + k * 8, 8)])
        return _
    lax.fori_loop(0, RPT // 8, zrow, None)
    plsc.subcore_barrier()

    def block(r, _):
        pltpu.sync_copy(dsts_hbm.at[cid, sid, r], dst_all)

        def chunk(i, __):
            pltpu.sync_copy(ones_v, acc_sh.at[dst_all.at[i]], add=True)
            return __
        lax.fori_loop(0, IBUF, chunk, None)
        return _
    lax.fori_loop(0, NRELOAD, block, None)

    plsc.subcore_barrier()
    pltpu.sync_copy(acc_sh.at[pl.ds(row0, RPT)],
                    deg_hbm.at[cid, pl.ds(row0, RPT)])


@functools.partial(
    pl.kernel,
    out_type=jax.ShapeDtypeStruct((NC, N, D), jnp.float32),
    mesh=_mesh,
    scratch_types=[
        pltpu.VMEM_SHARED((NA, D), jnp.float32),
        pltpu.VMEM((IBUF, CH), jnp.int32),
        pltpu.VMEM((IBUF, CH), jnp.int32),
        pltpu.VMEM((CH, D), jnp.float32),
        pltpu.VMEM((CH, D), jnp.float32),
        pltpu.VMEM((8, D), jnp.float32),
        pltpu.SemaphoreType.DMA,
        pltpu.SemaphoreType.DMA,
    ],
)
def _sc_edge_pass(g_hbm, srcs_hbm, dsts_hbm, out_hbm,
                  acc_sh, src_all, dst_all, rows_a, rows_b, zbuf,
                  sem_a, sem_b):
    cid = lax.axis_index("c")
    sid = lax.axis_index("s")
    row0 = jnp.minimum(sid * RPT, N - RPT)

    def initbuf(i, _):
        def initrow(j, __):
            zbuf[i, pl.ds(j * 16, 16)] = jnp.zeros((16,), jnp.float32)
            return __
        lax.fori_loop(0, D // 16, initrow, None)
        return _
    lax.fori_loop(0, 8, initbuf, None)

    def zrow(k, _):
        pltpu.sync_copy(zbuf, acc_sh.at[pl.ds(row0 + k * 8, 8)])
        return _
    lax.fori_loop(0, RPT // 8, zrow, None)
    plsc.subcore_barrier()

    def block(r, _):
        pltpu.sync_copy(srcs_hbm.at[cid, sid, r], src_all)
        pltpu.sync_copy(dsts_hbm.at[cid, sid, r], dst_all)
        pltpu.async_copy(g_hbm.at[src_all.at[0]], rows_a, sem_a)

        def pair(j, __):
            i0 = 2 * j
            pltpu.async_copy(g_hbm.at[src_all.at[i0 + 1]], rows_b, sem_b)
            pltpu.make_async_copy(g_hbm.at[src_all.at[i0]], rows_a, sem_a).wait()
            pltpu.sync_copy(rows_a, acc_sh.at[dst_all.at[i0]], add=True)
            pltpu.async_copy(
                g_hbm.at[src_all.at[jnp.minimum(i0 + 2, IBUF - 1)]], rows_a, sem_a)
            pltpu.make_async_copy(g_hbm.at[src_all.at[i0 + 1]], rows_b, sem_b).wait()
            pltpu.sync_copy(rows_b, acc_sh.at[dst_all.at[i0 + 1]], add=True)
            return __
        lax.fori_loop(0, IBUF // 2, pair, None)
        pltpu.make_async_copy(g_hbm.at[src_all.at[IBUF - 1]], rows_a, sem_a).wait()
        return _
    lax.fori_loop(0, NRELOAD, block, None)

    plsc.subcore_barrier()
    pltpu.sync_copy(acc_sh.at[pl.ds(row0, RPT)],
                    out_hbm.at[cid, pl.ds(row0, RPT)])


_BLK = 1000
_GRID = (2 * N) // _BLK


def _tc_prep_body(x_ref, w_ref, deg_ref, g_ref):
    dis = lax.rsqrt(deg_ref[:, 0:1] + 1.0)
    g_ref[...] = jnp.dot(x_ref[...], w_ref[...],
                         preferred_element_type=jnp.float32) * dis


_tc_prep = pl.pallas_call(
    _tc_prep_body,
    grid=(_GRID,),
    in_specs=[
        pl.BlockSpec((_BLK, D), lambda i: (i, 0)),
        pl.BlockSpec((D, D), lambda i: (0, 0)),
        pl.BlockSpec((_BLK, D), lambda i: (i, 0)),
    ],
    out_specs=pl.BlockSpec((_BLK, D), lambda i: (i, 0)),
    out_shape=jax.ShapeDtypeStruct((2 * N, D), jnp.float32),
)


def _tc_mid_body(acc_ref, g_ref, deg_ref, b_ref, w_ref, out_ref):
    dis = lax.rsqrt(deg_ref[:, 0:1] + 1.0)
    h = dis * (acc_ref[...] + g_ref[...]) + b_ref[...]
    h = jnp.maximum(h, 0.0)
    out_ref[...] = jnp.dot(h, w_ref[...],
                           preferred_element_type=jnp.float32) * dis


_tc_mid = pl.pallas_call(
    _tc_mid_body,
    grid=(_GRID,),
    in_specs=[
        pl.BlockSpec((_BLK, D), lambda i: (i, 0)),
        pl.BlockSpec((_BLK, D), lambda i: (i, 0)),
        pl.BlockSpec((_BLK, D), lambda i: (i, 0)),
        pl.BlockSpec((1, D), lambda i: (0, 0)),
        pl.BlockSpec((D, D), lambda i: (0, 0)),
    ],
    out_specs=pl.BlockSpec((_BLK, D), lambda i: (i, 0)),
    out_shape=jax.ShapeDtypeStruct((2 * N, D), jnp.float32),
)


def _tc_final_body(acc_ref, g_ref, deg_ref, b_ref, h_ref, s_ref, ss_ref):
    dis = lax.rsqrt(deg_ref[:, 0:1] + 1.0)
    h = dis * (acc_ref[...] + g_ref[...]) + b_ref[...]
    h_ref[...] = h
    s_ref[0, ...] = jnp.sum(h, axis=0, keepdims=True)
    ss_ref[0, ...] = jnp.sum(h * h, axis=0, keepdims=True)


_tc_final = pl.pallas_call(
    _tc_final_body,
    grid=(_GRID,),
    in_specs=[
        pl.BlockSpec((_BLK, D), lambda i: (i, 0)),
        pl.BlockSpec((_BLK, D), lambda i: (i, 0)),
        pl.BlockSpec((_BLK, D), lambda i: (i, 0)),
        pl.BlockSpec((1, D), lambda i: (0, 0)),
    ],
    out_specs=[
        pl.BlockSpec((_BLK, D), lambda i: (i, 0)),
        pl.BlockSpec((1, 1, D), lambda i: (i, 0, 0)),
        pl.BlockSpec((1, 1, D), lambda i: (i, 0, 0)),
    ],
    out_shape=[
        jax.ShapeDtypeStruct((2 * N, D), jnp.float32),
        jax.ShapeDtypeStruct((_GRID, 1, D), jnp.float32),
        jax.ShapeDtypeStruct((_GRID, 1, D), jnp.float32),
    ],
)


def _tc_norm_body(h_ref, s_ref, ss_ref, z_ref):
    gid = pl.program_id(0) // (_GRID // 2)
    rows = lax.broadcasted_iota(jnp.int32, (_GRID, 1, D), 0) // (_GRID // 2)
    mask = rows == gid
    s = jnp.sum(jnp.where(mask, s_ref[...], 0.0), axis=0)
    ss = jnp.sum(jnp.where(mask, ss_ref[...], 0.0), axis=0)
    n = jnp.float32(N)
    mean = s / n
    var = (ss - n * mean * mean) / (n - 1.0)
    z_ref[...] = (h_ref[...] - mean) * lax.rsqrt(var)


_tc_norm = pl.pallas_call(
    _tc_norm_body,
    grid=(_GRID,),
    in_specs=[
        pl.BlockSpec((_BLK, D), lambda i: (i, 0)),
        pl.BlockSpec((_GRID, 1, D), lambda i: (0, 0, 0)),
        pl.BlockSpec((_GRID, 1, D), lambda i: (0, 0, 0)),
    ],
    out_specs=pl.BlockSpec((_BLK, D), lambda i: (i, 0)),
    out_shape=jax.ShapeDtypeStruct((2 * N, D), jnp.float32),
)


def kernel(x1, edge_index1, x2, edge_index2, p1, p2, W1, b1, W2, b2):
    del p1, p2
    srcs = jnp.concatenate([edge_index1[0].astype(jnp.int32),
                            edge_index2[0].astype(jnp.int32) + N]
                           ).reshape(NC, NS, EPT)
    dsts = jnp.concatenate([edge_index1[1].astype(jnp.int32),
                            edge_index2[1].astype(jnp.int32)]
                           ).reshape(NC, NS, EPT)
    pad = ((0, 0), (0, 0), (0, EPT_P - EPT))
    srcs = jnp.pad(srcs, pad).reshape(NC, NS, NRELOAD, IBUF, CH)
    dsts = jnp.pad(dsts, pad, constant_values=NJUNK
                   ).reshape(NC, NS, NRELOAD, IBUF, CH)
    xs = jnp.concatenate([x1, x2], axis=0)
    b1r = b1.reshape(1, D)
    b2r = b2.reshape(1, D)

    deg = _sc_degree(dsts).reshape(2 * N, D)

    g1 = _tc_prep(xs, W1, deg)
    acc1 = _sc_edge_pass(g1, srcs, dsts).reshape(2 * N, D)
    g2 = _tc_mid(acc1, g1, deg, b1r, W2)
    acc2 = _sc_edge_pass(g2, srcs, dsts).reshape(2 * N, D)
    h, s, ss = _tc_final(acc2, g2, deg, b2r)
    z = _tc_norm(h, s, ss)
    return (z[:N], z[N:])


# speedup vs baseline: 3.0922x; 3.0922x over previous
import functools

import jax
import jax.numpy as jnp
from jax import lax
from jax.experimental import pallas as pl
from jax.experimental.pallas import tpu as pltpu
from jax.experimental.pallas import tpu_sc as plsc

N = 10000
D = 128
E = 320000

NC = 2
NS = 16
CH = 128
EPT = E // NS
EPT_P = 20480
NCHUNK = EPT_P // CH
RPT = 632
NRELOAD = 5
IBUF = NCHUNK // NRELOAD
NJUNK = N
NA = N + 16

_mesh = plsc.VectorSubcoreMesh(core_axis_name="c", subcore_axis_name="s")


@functools.partial(
    pl.kernel,
    out_type=jax.ShapeDtypeStruct((NC, N, D), jnp.float32),
    mesh=_mesh,
    scratch_types=[
        pltpu.VMEM_SHARED((NA, D), jnp.float32),
        pltpu.VMEM((IBUF, CH), jnp.int32),
        pltpu.VMEM((CH, D), jnp.float32),
        pltpu.VMEM((8, D), jnp.float32),
    ],
)
def _sc_degree(dsts_hbm, deg_hbm, acc_sh, dst_all, ones_v, zbuf):
    cid = lax.axis_index("c")
    sid = lax.axis_index("s")
    row0 = jnp.minimum(sid * RPT, N - RPT)

    def initbuf(i, _):
        def initrow(j, __):
            zbuf[i, pl.ds(j * 16, 16)] = jnp.zeros((16,), jnp.float32)
            return __
        lax.fori_loop(0, D // 16, initrow, None)
        return _
    lax.fori_loop(0, 8, initbuf, None)

    def initones(i, _):
        def initrow(j, __):
            ones_v[i, pl.ds(j * 16, 16)] = jnp.ones((16,), jnp.float32)
            return __
        lax.fori_loop(0, D // 16, initrow, None)
        return _
    lax.fori_loop(0, CH, initones, None)

    def zrow(k, _):
        pltpu.sync_copy(zbuf, acc_sh.at[pl.ds(row0 + k * 8, 8)])
        return _
    lax.fori_loop(0, RPT // 8, zrow, None)
    plsc.subcore_barrier()

    def block(r, _):
        pltpu.sync_copy(dsts_hbm.at[cid, sid, r], dst_all)

        def chunk(i, __):
            pltpu.sync_copy(ones_v, acc_sh.at[dst_all.at[i]], add=True)
            return __
        lax.fori_loop(0, IBUF, chunk, None)
        return _
    lax.fori_loop(0, NRELOAD, block, None)

    plsc.subcore_barrier()
    pltpu.sync_copy(acc_sh.at[pl.ds(row0, RPT)],
                    deg_hbm.at[cid, pl.ds(row0, RPT)])


@functools.partial(
    pl.kernel,
    out_type=jax.ShapeDtypeStruct((NC, N, D), jnp.float32),
    mesh=_mesh,
    scratch_types=[
        pltpu.VMEM_SHARED((NA, D), jnp.float32),
        pltpu.VMEM((IBUF, CH), jnp.int32),
        pltpu.VMEM((IBUF, CH), jnp.int32),
        pltpu.VMEM((CH, D), jnp.float32),
        pltpu.VMEM((CH, D), jnp.float32),
        pltpu.VMEM((8, D), jnp.float32),
        pltpu.SemaphoreType.DMA,
        pltpu.SemaphoreType.DMA,
    ],
)
def _sc_edge_pass(g_hbm, srcs_hbm, dsts_hbm, out_hbm,
                  acc_sh, src_all, dst_all, rows_a, rows_b, zbuf,
                  sem_a, sem_b):
    cid = lax.axis_index("c")
    sid = lax.axis_index("s")
    row0 = jnp.minimum(sid * RPT, N - RPT)

    def initbuf(i, _):
        def initrow(j, __):
            zbuf[i, pl.ds(j * 16, 16)] = jnp.zeros((16,), jnp.float32)
            return __
        lax.fori_loop(0, D // 16, initrow, None)
        return _
    lax.fori_loop(0, 8, initbuf, None)

    def zrow(k, _):
        pltpu.sync_copy(zbuf, acc_sh.at[pl.ds(row0 + k * 8, 8)])
        return _
    lax.fori_loop(0, RPT // 8, zrow, None)
    plsc.subcore_barrier()

    def block(r, _):
        pltpu.sync_copy(srcs_hbm.at[cid, sid, r], src_all)
        pltpu.sync_copy(dsts_hbm.at[cid, sid, r], dst_all)
        pltpu.async_copy(g_hbm.at[src_all.at[0]], rows_a, sem_a)

        def pair(j, __):
            i0 = 2 * j
            pltpu.async_copy(g_hbm.at[src_all.at[i0 + 1]], rows_b, sem_b)
            pltpu.make_async_copy(g_hbm.at[src_all.at[i0]], rows_a, sem_a).wait()
            pltpu.sync_copy(rows_a, acc_sh.at[dst_all.at[i0]], add=True)
            pltpu.async_copy(
                g_hbm.at[src_all.at[jnp.minimum(i0 + 2, IBUF - 1)]], rows_a, sem_a)
            pltpu.make_async_copy(g_hbm.at[src_all.at[i0 + 1]], rows_b, sem_b).wait()
            pltpu.sync_copy(rows_b, acc_sh.at[dst_all.at[i0 + 1]], add=True)
            return __
        lax.fori_loop(0, IBUF // 2, pair, None)
        pltpu.make_async_copy(g_hbm.at[src_all.at[IBUF - 1]], rows_a, sem_a).wait()
        return _
    lax.fori_loop(0, NRELOAD, block, None)

    plsc.subcore_barrier()
    pltpu.sync_copy(acc_sh.at[pl.ds(row0, RPT)],
                    out_hbm.at[cid, pl.ds(row0, RPT)])


_BLK = 1000
_GRID = (2 * N) // _BLK


def _tc_prep_body(x_ref, w_ref, deg_ref, g_ref):
    dis = lax.rsqrt(deg_ref[:, 0:1] + 1.0)
    g_ref[...] = jnp.dot(x_ref[...], w_ref[...],
                         preferred_element_type=jnp.float32) * dis


_tc_prep = pl.pallas_call(
    _tc_prep_body,
    grid=(_GRID,),
    in_specs=[
        pl.BlockSpec((_BLK, D), lambda i: (i, 0)),
        pl.BlockSpec((D, D), lambda i: (0, 0)),
        pl.BlockSpec((_BLK, D), lambda i: (i, 0)),
    ],
    out_specs=pl.BlockSpec((_BLK, D), lambda i: (i, 0)),
    out_shape=jax.ShapeDtypeStruct((2 * N, D), jnp.float32),
)


def _tc_mid_body(acc_ref, g_ref, deg_ref, b_ref, w_ref, out_ref):
    dis = lax.rsqrt(deg_ref[:, 0:1] + 1.0)
    h = dis * (acc_ref[...] + g_ref[...]) + b_ref[...]
    h = jnp.maximum(h, 0.0)
    out_ref[...] = jnp.dot(h, w_ref[...],
                           preferred_element_type=jnp.float32) * dis


_tc_mid = pl.pallas_call(
    _tc_mid_body,
    grid=(_GRID,),
    in_specs=[
        pl.BlockSpec((_BLK, D), lambda i: (i, 0)),
        pl.BlockSpec((_BLK, D), lambda i: (i, 0)),
        pl.BlockSpec((_BLK, D), lambda i: (i, 0)),
        pl.BlockSpec((1, D), lambda i: (0, 0)),
        pl.BlockSpec((D, D), lambda i: (0, 0)),
    ],
    out_specs=pl.BlockSpec((_BLK, D), lambda i: (i, 0)),
    out_shape=jax.ShapeDtypeStruct((2 * N, D), jnp.float32),
)


def _tc_final_body(acc_ref, g_ref, deg_ref, b_ref, h_ref, s_ref, ss_ref):
    dis = lax.rsqrt(deg_ref[:, 0:1] + 1.0)
    h = dis * (acc_ref[...] + g_ref[...]) + b_ref[...]
    h_ref[...] = h
    s_ref[0, ...] = jnp.sum(h, axis=0, keepdims=True)
    ss_ref[0, ...] = jnp.sum(h * h, axis=0, keepdims=True)


_tc_final = pl.pallas_call(
    _tc_final_body,
    grid=(_GRID,),
    in_specs=[
        pl.BlockSpec((_BLK, D), lambda i: (i, 0)),
        pl.BlockSpec((_BLK, D), lambda i: (i, 0)),
        pl.BlockSpec((_BLK, D), lambda i: (i, 0)),
        pl.BlockSpec((1, D), lambda i: (0, 0)),
    ],
    out_specs=[
        pl.BlockSpec((_BLK, D), lambda i: (i, 0)),
        pl.BlockSpec((1, 1, D), lambda i: (i, 0, 0)),
        pl.BlockSpec((1, 1, D), lambda i: (i, 0, 0)),
    ],
    out_shape=[
        jax.ShapeDtypeStruct((2 * N, D), jnp.float32),
        jax.ShapeDtypeStruct((_GRID, 1, D), jnp.float32),
        jax.ShapeDtypeStruct((_GRID, 1, D), jnp.float32),
    ],
)


def _tc_norm_body(h_ref, s_ref, ss_ref, z_ref):
    gid = pl.program_id(0) // (_GRID // 2)
    rows = lax.broadcasted_iota(jnp.int32, (_GRID, 1, D), 0) // (_GRID // 2)
    mask = rows == gid
    s = jnp.sum(jnp.where(mask, s_ref[...], 0.0), axis=0)
    ss = jnp.sum(jnp.where(mask, ss_ref[...], 0.0), axis=0)
    n = jnp.float32(N)
    mean = s / n
    var = (ss - n * mean * mean) / (n - 1.0)
    z_ref[...] = (h_ref[...] - mean) * lax.rsqrt(var)


_tc_norm = pl.pallas_call(
    _tc_norm_body,
    grid=(_GRID,),
    in_specs=[
        pl.BlockSpec((_BLK, D), lambda i: (i, 0)),
        pl.BlockSpec((_GRID, 1, D), lambda i: (0, 0, 0)),
        pl.BlockSpec((_GRID, 1, D), lambda i: (0, 0, 0)),
    ],
    out_specs=pl.BlockSpec((_BLK, D), lambda i: (i, 0)),
    out_shape=jax.ShapeDtypeStruct((2 * N, D), jnp.float32),
)


def kernel(x1, edge_index1, x2, edge_index2, p1, p2, W1, b1, W2, b2):
    del p1, p2
    srcs = jnp.concatenate([edge_index1[0].astype(jnp.int32),
                            edge_index2[0].astype(jnp.int32) + N]
                           ).reshape(NC, NS, EPT)
    dsts = jnp.concatenate([edge_index1[1].astype(jnp.int32),
                            edge_index2[1].astype(jnp.int32)]
                           ).reshape(NC, NS, EPT)
    npad = EPT_P - EPT
    pad_src = jnp.broadcast_to(
        (jnp.arange(npad, dtype=jnp.int32) * 137) % (2 * N), (NC, NS, npad))
    pad_dst = jnp.broadcast_to(
        NJUNK + (jnp.arange(npad, dtype=jnp.int32) % 16), (NC, NS, npad))
    srcs = jnp.concatenate([srcs, pad_src], axis=2
                           ).reshape(NC, NS, NRELOAD, IBUF, CH)
    dsts = jnp.concatenate([dsts, pad_dst], axis=2
                           ).reshape(NC, NS, NRELOAD, IBUF, CH)
    xs = jnp.concatenate([x1, x2], axis=0)
    b1r = b1.reshape(1, D)
    b2r = b2.reshape(1, D)

    deg = _sc_degree(dsts).reshape(2 * N, D)

    g1 = _tc_prep(xs, W1, deg)
    acc1 = _sc_edge_pass(g1, srcs, dsts).reshape(2 * N, D)
    g2 = _tc_mid(acc1, g1, deg, b1r, W2)
    acc2 = _sc_edge_pass(g2, srcs, dsts).reshape(2 * N, D)
    h, s, ss = _tc_final(acc2, g2, deg, b2r)
    z = _tc_norm(h, s, ss)
    return (z[:N], z[N:])
